# Initial kernel scaffold; baseline (speedup 1.0000x reference)
import jax
import jax.numpy as jnp
from jax.experimental import pallas as pl


def kernel(x, edge_index0, edge_index1, edge_index2, edge_index3, edge_index4, pool1_ids, pool2_ids, pool3_ids, pool4_ids, Wn1, Ws1, g1, b1, Wn2, Ws2, g2, b2, Wn3, Ws3, g3, b3, Wn4, Ws4, g4, b4, Wn5, Ws5, g5, b5, Wn6, Ws6, g6, b6, Wn7, Ws7, g7, b7, Wn8, Ws8, g8, b8, Wn9, Ws9, g9, b9, Wn10, Ws10, g10, b10, Wn11, Ws11, g11, b11, Wn12, Ws12, g12, b12, Wn13, Ws13, g13, b13):
    raise NotImplementedError("write your pallas kernel here")


# speedup vs baseline: 1.0126x; 1.0126x over previous
import functools

import jax
import jax.numpy as jnp
from jax import lax
from jax.experimental import pallas as pl
from jax.experimental.pallas import tpu as pltpu
from jax.experimental.pallas import tpu_sc as plsc

_NODES = [10000, 2500, 625, 160, 40]


def _layer_body(agg_ref, h_ref, wn_ref, ws_ref, g_ref, b_ref, o_ref):
    z = jnp.dot(agg_ref[...], wn_ref[...], preferred_element_type=jnp.float32)
    z = z + jnp.dot(h_ref[...], ws_ref[...], preferred_element_type=jnp.float32)
    mu = jnp.mean(z, axis=0, keepdims=True)
    zc = z - mu
    var = jnp.mean(zc * zc, axis=0, keepdims=True)
    zn = zc * lax.rsqrt(var + 1e-5) * g_ref[...] + b_ref[...]
    o_ref[...] = jnp.maximum(zn, 0.0)


def _layer(agg, h, wn, ws, g, b):
    n, cout = h.shape[0], wn.shape[1]
    return pl.pallas_call(
        _layer_body,
        out_shape=jax.ShapeDtypeStruct((n, cout), jnp.float32),
    )(agg, h, wn, ws, g.reshape(1, -1), b.reshape(1, -1))


def _emb_body(o2_ref, o3_ref, o4_ref, out_ref):
    m2 = jnp.mean(o2_ref[...], axis=0, keepdims=True)
    m3 = jnp.mean(o3_ref[...], axis=0, keepdims=True)
    m4 = jnp.mean(o4_ref[...], axis=0, keepdims=True)
    ss = jnp.sum(m2 * m2) + jnp.sum(m3 * m3) + jnp.sum(m4 * m4)
    inv = 1.0 / jnp.maximum(jnp.sqrt(ss), 1e-12)
    out_ref[0:1, 0:128] = m2 * inv
    out_ref[0:1, 128:384] = m3 * inv
    out_ref[0:1, 384:896] = m4 * inv


def _embed(o2, o3, o4):
    return pl.pallas_call(
        _emb_body,
        out_shape=jax.ShapeDtypeStruct((1, 896), jnp.float32),
    )(o2, o3, o4)


def _segsum(h, src, dst, n_out):
    return jax.ops.segment_sum(h[src], dst, num_segments=n_out)


def _pool(h, ids, n_out):
    y = jax.ops.segment_max(h, ids, num_segments=n_out)
    return jnp.where(jnp.isfinite(y), y, 0.0)


def kernel(x, edge_index0, edge_index1, edge_index2, edge_index3, edge_index4,
           pool1_ids, pool2_ids, pool3_ids, pool4_ids,
           Wn1, Ws1, g1, b1, Wn2, Ws2, g2, b2, Wn3, Ws3, g3, b3,
           Wn4, Ws4, g4, b4, Wn5, Ws5, g5, b5, Wn6, Ws6, g6, b6,
           Wn7, Ws7, g7, b7, Wn8, Ws8, g8, b8, Wn9, Ws9, g9, b9,
           Wn10, Ws10, g10, b10, Wn11, Ws11, g11, b11,
           Wn12, Ws12, g12, b12, Wn13, Ws13, g13, b13):
    p = locals()
    e = [p['edge_index%d' % i] for i in range(5)]

    def layer(h, i, ei):
        agg = _segsum(h, ei[0], ei[1], h.shape[0])
        return _layer(agg, h, p['Wn%d' % i], p['Ws%d' % i], p['g%d' % i], p['b%d' % i])

    h = layer(x, 1, e[0])
    h = layer(h, 2, e[0])
    h = _pool(h, pool1_ids, _NODES[1])
    h = layer(h, 3, e[1])
    out2 = layer(h, 4, e[1])
    h = _pool(out2, pool2_ids, _NODES[2])
    h = layer(h, 5, e[2])
    h = layer(h, 6, e[2])
    h = layer(h, 7, e[2])
    out3 = _pool(h, pool3_ids, _NODES[3])
    h = layer(out3, 8, e[3])
    h = layer(h, 9, e[3])
    h = layer(h, 10, e[3])
    h = _pool(h, pool4_ids, _NODES[4])
    h = layer(h, 11, e[4])
    h = layer(h, 12, e[4])
    out4 = layer(h, 13, e[4])
    return _embed(out2, out3, out4)


# speedup vs baseline: 4.3284x; 4.2745x over previous
import functools

import jax
import jax.numpy as jnp
from jax import lax
from jax.experimental import pallas as pl
from jax.experimental.pallas import tpu as pltpu
from jax.experimental.pallas import tpu_sc as plsc

_NODES = [10000, 2500, 625, 160, 40]

_NC, _NS, _LANES = 2, 16, 16
_NW = _NC * _NS
_K = 128


def _cdiv(a, b):
    return -(-a // b)


def _segsum_sc(h, src, dst, n_nodes):
    if h.shape[1] % 128 != 0:
        h = jnp.pad(h, ((0, 0), (0, 128 - h.shape[1] % 128)))
    C = h.shape[1]
    E = src.shape[0]
    Epad = _cdiv(E, _K) * _K
    if Epad != E:
        src = jnp.pad(src, (0, Epad - E))
        dst = jnp.pad(dst, (0, Epad - E), constant_values=n_nodes)
    ZR = max(8, min(64, 8192 // C))
    Npad = _cdiv(n_nodes + 1, _NS * ZR) * (_NS * ZR)
    slab = Npad // _NS
    nchunks = Epad // _K
    iters = _cdiv(nchunks, _NW)

    mesh = plsc.VectorSubcoreMesh(core_axis_name="c", subcore_axis_name="s")

    @functools.partial(
        pl.kernel, mesh=mesh,
        out_type=jax.ShapeDtypeStruct((_NC, Npad, C), jnp.float32),
        scratch_types=[
            pltpu.VMEM((_K,), jnp.int32),
            pltpu.VMEM((_K,), jnp.int32),
            pltpu.VMEM((_K, C), jnp.float32),
            pltpu.VMEM((ZR, C), jnp.float32),
            pltpu.VMEM_SHARED((Npad, C), jnp.float32),
            pltpu.SemaphoreType.DMA,
        ],
        compiler_params=pltpu.CompilerParams(use_tc_tiling_on_sc=False),
    )
    def k(src_h, dst_h, h_h, out_h, idx_s, idx_d, rows, zrow, acc, sem):
        cid = lax.axis_index("c")
        sid = lax.axis_index("s")
        wid = sid * _NC + cid
        zero = jnp.zeros((_LANES,), jnp.float32)
        for r in range(ZR):
            for j in range(C // _LANES):
                zrow[r, pl.ds(j * _LANES, _LANES)] = zero

        @pl.when(sid * slab < n_nodes)
        def _():
            for t in range(slab // ZR):
                pltpu.sync_copy(zrow, acc.at[pl.ds(sid * slab + t * ZR, ZR)])

        plsc.subcore_barrier()

        def body(it, carry):
            c = it * _NW + wid

            @pl.when(c < nchunks)
            def _():
                base = c * _K
                pltpu.sync_copy(src_h.at[pl.ds(base, _K)], idx_s)
                pltpu.sync_copy(dst_h.at[pl.ds(base, _K)], idx_d)
                pltpu.async_copy(h_h.at[idx_s], rows, sem).wait()
                pltpu.sync_copy(rows, acc.at[idx_d], add=True)

            return carry

        lax.fori_loop(0, iters, body, 0)
        plsc.subcore_barrier()

        @pl.when(sid * slab < n_nodes)
        def _():
            pltpu.sync_copy(acc.at[pl.ds(sid * slab, slab)],
                            out_h.at[cid, pl.ds(sid * slab, slab)])

    return k(src, dst, h)


def _layer_body(n, c_in, aggp_ref, h_ref, wn_ref, ws_ref, g_ref, b_ref, o_ref):
    agg = aggp_ref[0, :n, :c_in] + aggp_ref[1, :n, :c_in]
    z = jnp.dot(agg, wn_ref[...], preferred_element_type=jnp.float32)
    z = z + jnp.dot(h_ref[...], ws_ref[...], preferred_element_type=jnp.float32)
    mu = jnp.mean(z, axis=0, keepdims=True)
    zc = z - mu
    var = jnp.mean(zc * zc, axis=0, keepdims=True)
    zn = zc * lax.rsqrt(var + 1e-5) * g_ref[...] + b_ref[...]
    o_ref[...] = jnp.maximum(zn, 0.0)


def _layer(aggp, h, wn, ws, g, b):
    n, cout = h.shape[0], wn.shape[1]
    return pl.pallas_call(
        functools.partial(_layer_body, n, wn.shape[0]),
        out_shape=jax.ShapeDtypeStruct((n, cout), jnp.float32),
    )(aggp, h, wn, ws, g.reshape(1, -1), b.reshape(1, -1))


def _emb_body(o2_ref, o3_ref, o4_ref, out_ref):
    m2 = jnp.mean(o2_ref[...], axis=0, keepdims=True)
    m3 = jnp.mean(o3_ref[...], axis=0, keepdims=True)
    m4 = jnp.mean(o4_ref[...], axis=0, keepdims=True)
    ss = jnp.sum(m2 * m2) + jnp.sum(m3 * m3) + jnp.sum(m4 * m4)
    inv = 1.0 / jnp.maximum(jnp.sqrt(ss), 1e-12)
    out_ref[0:1, 0:128] = m2 * inv
    out_ref[0:1, 128:384] = m3 * inv
    out_ref[0:1, 384:896] = m4 * inv


def _embed(o2, o3, o4):
    return pl.pallas_call(
        _emb_body,
        out_shape=jax.ShapeDtypeStruct((1, 896), jnp.float32),
    )(o2, o3, o4)


def _pool(h, ids, n_out):
    y = jax.ops.segment_max(h, ids, num_segments=n_out)
    return jnp.where(jnp.isfinite(y), y, 0.0)


def kernel(x, edge_index0, edge_index1, edge_index2, edge_index3, edge_index4,
           pool1_ids, pool2_ids, pool3_ids, pool4_ids,
           Wn1, Ws1, g1, b1, Wn2, Ws2, g2, b2, Wn3, Ws3, g3, b3,
           Wn4, Ws4, g4, b4, Wn5, Ws5, g5, b5, Wn6, Ws6, g6, b6,
           Wn7, Ws7, g7, b7, Wn8, Ws8, g8, b8, Wn9, Ws9, g9, b9,
           Wn10, Ws10, g10, b10, Wn11, Ws11, g11, b11,
           Wn12, Ws12, g12, b12, Wn13, Ws13, g13, b13):
    p = locals()
    e = [p['edge_index%d' % i] for i in range(5)]

    def layer(h, i, ei):
        aggp = _segsum_sc(h, ei[0], ei[1], h.shape[0])
        return _layer(aggp, h, p['Wn%d' % i], p['Ws%d' % i], p['g%d' % i], p['b%d' % i])

    h = layer(x, 1, e[0])
    h = layer(h, 2, e[0])
    h = _pool(h, pool1_ids, _NODES[1])
    h = layer(h, 3, e[1])
    out2 = layer(h, 4, e[1])
    h = _pool(out2, pool2_ids, _NODES[2])
    h = layer(h, 5, e[2])
    h = layer(h, 6, e[2])
    h = layer(h, 7, e[2])
    out3 = _pool(h, pool3_ids, _NODES[3])
    h = layer(out3, 8, e[3])
    h = layer(h, 9, e[3])
    h = layer(h, 10, e[3])
    h = _pool(h, pool4_ids, _NODES[4])
    h = layer(h, 11, e[4])
    h = layer(h, 12, e[4])
    out4 = layer(h, 13, e[4])
    return _embed(out2, out3, out4)
